# baseline (speedup 1.0000x reference)
import functools

import jax
import jax.numpy as jnp
from jax import lax
from jax.experimental import pallas as pl
from jax.experimental.pallas import tpu as pltpu
from jax.experimental.pallas import tpu_sc as plsc

N = 3276800
NC = 2
NS = 16
L = 16
NW = NC * NS

N_SC = 1802240
N_TC = N - N_SC
PER_W = N_SC // NW
CHUNK = PER_W // 8
NCH = 8
U = 8

COLS = 128
ROWS = N // COLS
SC_ROWS = N_SC // COLS
TC_BLOCK_ROWS = 1280
TC_GRID = (ROWS - SC_ROWS) // TC_BLOCK_ROWS
SC_BLOCKS = SC_ROWS // TC_BLOCK_ROWS

_mesh = plsc.VectorSubcoreMesh(core_axis_name="c", subcore_axis_name="s")


@functools.partial(
    pl.kernel,
    mesh=_mesh,
    out_type=jax.ShapeDtypeStruct((NW, L), jnp.int32),
    scratch_types=[
        pltpu.VMEM((PER_W,), jnp.int32),
        pltpu.VMEM((L,), jnp.int32),
    ] + [pltpu.SemaphoreType.DMA] * NCH,
)
def _sc_count(in_hbm, out_hbm, buf_v, part_v, *sems):
    wid = lax.axis_index("s") * NC + lax.axis_index("c")
    base = wid * PER_W
    handles = [
        pltpu.async_copy(
            in_hbm.at[pl.ds(base + g * CHUNK, CHUNK)],
            buf_v.at[pl.ds(g * CHUNK, CHUNK)],
            sems[g],
        )
        for g in range(NCH)
    ]
    accs = tuple(jnp.zeros((L,), jnp.int32) for _ in range(U))
    for g in range(NCH):
        handles[g].wait()

        def body(i, accs_t, goff=g * CHUNK):
            off = goff + i * (U * L)
            return tuple(accs_t[u] + buf_v[pl.ds(off + u * L, L)] for u in range(U))

        accs = lax.fori_loop(0, CHUNK // (U * L), body, accs)
    acc = accs[0]
    for u in range(1, U):
        acc = acc + accs[u]
    part_v[...] = acc
    pltpu.sync_copy(part_v, out_hbm.at[wid])


def _tc_reduce_body(x_ref, o_ref):
    @pl.when(pl.program_id(0) == 0)
    def _init():
        o_ref[...] = jnp.zeros((8, COLS), jnp.int32)

    x = x_ref[...].reshape(TC_BLOCK_ROWS // 8, 8, COLS)
    o_ref[...] += jnp.sum(x, axis=0)


_tc_reduce = pl.pallas_call(
    _tc_reduce_body,
    grid=(TC_GRID,),
    in_specs=[pl.BlockSpec((TC_BLOCK_ROWS, COLS), lambda i: (SC_BLOCKS + i, 0))],
    out_specs=pl.BlockSpec((8, COLS), lambda i: (0, 0)),
    out_shape=jax.ShapeDtypeStruct((8, COLS), jnp.int32),
)


def _combine_body(part_ref, t_ref, w_ref, o_ref):
    s = (jnp.sum(part_ref[...]) + jnp.sum(t_ref[...])).astype(jnp.float32)
    frac = s * (1.0 / N)
    w = w_ref[...]
    o_ref[...] = (1.0 - frac) * w[0:1, :] + frac * w[1:2, :]


def _combine(partials, tc_sum, weight):
    return pl.pallas_call(
        _combine_body,
        in_specs=[
            pl.BlockSpec(memory_space=pltpu.VMEM),
            pl.BlockSpec(memory_space=pltpu.VMEM),
            pl.BlockSpec(memory_space=pltpu.VMEM),
        ],
        out_shape=jax.ShapeDtypeStruct((1, 3), jnp.float32),
    )(partials, tc_sum, weight)


def kernel(input, weight):
    idx = input if input.dtype == jnp.int32 else input.astype(jnp.int32)
    sc_partials = _sc_count(idx)
    tc_sum = _tc_reduce(idx.reshape(ROWS, COLS))
    return _combine(sc_partials, tc_sum, weight)


# speedup vs baseline: 1.0197x; 1.0197x over previous
import functools

import jax
import jax.numpy as jnp
from jax import lax
from jax.experimental import pallas as pl
from jax.experimental.pallas import tpu as pltpu
from jax.experimental.pallas import tpu_sc as plsc

N = 3276800
NC = 2
NS = 16
L = 16
NW = NC * NS

N_SC = 1966080
N_TC = N - N_SC
PER_W = N_SC // NW
CHUNK = PER_W // 8
NCH = 8
U = 8

COLS = 128
ROWS = N // COLS
SC_ROWS = N_SC // COLS
TC_BLOCK_ROWS = 2560
TC_GRID = (ROWS - SC_ROWS) // TC_BLOCK_ROWS
SC_BLOCKS = SC_ROWS // TC_BLOCK_ROWS

_mesh = plsc.VectorSubcoreMesh(core_axis_name="c", subcore_axis_name="s")


@functools.partial(
    pl.kernel,
    mesh=_mesh,
    out_type=jax.ShapeDtypeStruct((NW, L), jnp.int32),
    scratch_types=[
        pltpu.VMEM((PER_W,), jnp.int32),
        pltpu.VMEM((L,), jnp.int32),
    ] + [pltpu.SemaphoreType.DMA] * NCH,
)
def _sc_count(in_hbm, out_hbm, buf_v, part_v, *sems):
    wid = lax.axis_index("s") * NC + lax.axis_index("c")
    base = wid * PER_W
    handles = [
        pltpu.async_copy(
            in_hbm.at[pl.ds(base + g * CHUNK, CHUNK)],
            buf_v.at[pl.ds(g * CHUNK, CHUNK)],
            sems[g],
        )
        for g in range(NCH)
    ]
    accs = tuple(jnp.zeros((L,), jnp.int32) for _ in range(U))
    for g in range(NCH):
        handles[g].wait()

        def body(i, accs_t, goff=g * CHUNK):
            off = goff + i * (U * L)
            return tuple(accs_t[u] + buf_v[pl.ds(off + u * L, L)] for u in range(U))

        accs = lax.fori_loop(0, CHUNK // (U * L), body, accs)
    acc = accs[0]
    for u in range(1, U):
        acc = acc + accs[u]
    part_v[...] = acc
    pltpu.sync_copy(part_v, out_hbm.at[wid])


def _tc_reduce_body(x_ref, o_ref):
    @pl.when(pl.program_id(0) == 0)
    def _init():
        o_ref[...] = jnp.zeros((8, COLS), jnp.int32)

    x = x_ref[...].reshape(TC_BLOCK_ROWS // 8, 8, COLS)
    o_ref[...] += jnp.sum(x, axis=0)


_tc_reduce = pl.pallas_call(
    _tc_reduce_body,
    grid=(TC_GRID,),
    in_specs=[pl.BlockSpec((TC_BLOCK_ROWS, COLS), lambda i: (SC_BLOCKS + i, 0))],
    out_specs=pl.BlockSpec((8, COLS), lambda i: (0, 0)),
    out_shape=jax.ShapeDtypeStruct((8, COLS), jnp.int32),
)


def _combine_body(part_ref, t_ref, w_ref, o_ref):
    s = (jnp.sum(part_ref[...]) + jnp.sum(t_ref[...])).astype(jnp.float32)
    frac = s * (1.0 / N)
    w = w_ref[...]
    o_ref[...] = (1.0 - frac) * w[0:1, :] + frac * w[1:2, :]


def _combine(partials, tc_sum, weight):
    return pl.pallas_call(
        _combine_body,
        in_specs=[
            pl.BlockSpec(memory_space=pltpu.VMEM),
            pl.BlockSpec(memory_space=pltpu.VMEM),
            pl.BlockSpec(memory_space=pltpu.VMEM),
        ],
        out_shape=jax.ShapeDtypeStruct((1, 3), jnp.float32),
    )(partials, tc_sum, weight)


def kernel(input, weight):
    idx = input if input.dtype == jnp.int32 else input.astype(jnp.int32)
    sc_partials = _sc_count(idx)
    tc_sum = _tc_reduce(idx.reshape(ROWS, COLS))
    return _combine(sc_partials, tc_sum, weight)
